# baseline (speedup 1.0000x reference)
import functools

import jax
import jax.numpy as jnp
from jax import lax
from jax.experimental import pallas as pl
from jax.experimental.pallas import tpu as pltpu
from jax.experimental.pallas import tpu_sc as plsc

DIM = 32
LEVELS = 4
H0, W0 = 64, 256
NC, NS, L = 2, 16, 16
NW = NC * NS
CHUNK = 64
NTAP = 4 * LEVELS
NDMA = NTAP * CHUNK // 128

BH, BW = 64, 256
_BCELLS = BH * BW
_BLOCKS = [(H0 << l) // BH * ((W0 << l) // BW) for l in range(LEVELS)]
_STARTS = [sum(_BLOCKS[:l]) for l in range(LEVELS)]
_TOTAL_BLOCKS = sum(_BLOCKS)
_ROWS = _TOTAL_BLOCKS * _BCELLS
_Q = _BCELLS // 4
_LBH = BH.bit_length() - 1
_LBW = BW.bit_length() - 1
_LBC = _BCELLS.bit_length() - 1
_LQ = _Q.bit_length() - 1


def _tr_body(g0, g1, g2, g3, out_ref):
    p = pl.program_id(0)
    a = jnp.where(
        p < _STARTS[1], g0[...],
        jnp.where(p < _STARTS[2], g1[...],
                  jnp.where(p < _STARTS[3], g2[...], g3[...])))
    a2 = a.astype(jnp.bfloat16).reshape(DIM, _BCELLS)
    ai = a2.reshape(2, 16, _BCELLS).transpose(1, 0, 2).reshape(DIM, _BCELLS)
    out_ref[...] = jnp.concatenate(
        [ai[:, j * _Q:(j + 1) * _Q].T for j in range(4)], axis=1)


def _in_spec(l):
    nwb = (W0 << l) // BW

    def imap(p):
        q = jnp.clip(p - _STARTS[l], 0, _BLOCKS[l] - 1)
        return (0, q // nwb, q % nwb)

    return pl.BlockSpec((DIM, BH, BW), imap)


def _build_table(grid0, grid1, grid2, grid3):
    out = pl.pallas_call(
        _tr_body,
        grid=(_TOTAL_BLOCKS,),
        in_specs=[_in_spec(l) for l in range(LEVELS)],
        out_specs=pl.BlockSpec((_Q, 128), lambda p: (p, 0)),
        out_shape=jax.ShapeDtypeStruct((_TOTAL_BLOCKS * _Q, 128),
                                       jnp.bfloat16),
    )(grid0[0], grid1[0], grid2[0], grid3[0])
    ti = jax.lax.bitcast_convert_type(
        out.reshape(out.shape[0], 64, 2), jnp.int32)
    return ti.reshape(_ROWS, DIM // 2)


def _sc_sample(tsf, rhof, table):
    P = tsf.shape[0]
    ppw = P // NW
    nchunks = ppw // CHUNK
    mesh = plsc.VectorSubcoreMesh(core_axis_name="c", subcore_axis_name="s")

    @functools.partial(
        pl.kernel,
        out_type=jax.ShapeDtypeStruct((P, LEVELS * DIM), jnp.float32),
        mesh=mesh,
        scratch_types=[
            pltpu.VMEM((CHUNK,), jnp.float32),
            pltpu.VMEM((CHUNK,), jnp.float32),
            pltpu.VMEM((NTAP * CHUNK,), jnp.int32),
            pltpu.VMEM((NTAP * CHUNK,), jnp.int32),
            pltpu.VMEM((NTAP * CHUNK,), jnp.float32),
            pltpu.VMEM((NTAP * CHUNK,), jnp.float32),
            pltpu.VMEM((NTAP * CHUNK, DIM // 2), jnp.int32),
            pltpu.VMEM((NTAP * CHUNK, DIM // 2), jnp.int32),
            pltpu.VMEM((CHUNK, LEVELS * DIM), jnp.float32),
            pltpu.VMEM((CHUNK, LEVELS * DIM), jnp.float32),
            pltpu.SemaphoreType.DMA,
            pltpu.SemaphoreType.DMA,
        ],
        compiler_params=pltpu.CompilerParams(use_tc_tiling_on_sc=False),
    )
    def k(ts_hbm, rho_hbm, tab_hbm, out_hbm,
          ts_v, rho_v, idx_a, idx_b, w_a, w_b, rows_a, rows_b,
          out_a, out_b, sem_a, sem_b):
        wid = lax.axis_index("s") * NC + lax.axis_index("c")

        lane = lax.iota(jnp.int32, L)
        tvec = lane & 3
        lvec = lane >> 2
        wl_i = W0 << lvec
        hl_i = H0 << lvec
        wm1_f = (wl_i - 1).astype(jnp.float32)
        hm1_f = (hl_i - 1).astype(jnp.float32)
        wm2_i = wl_i - 2
        hm2_i = hl_i - 2
        tap_dx = tvec & 1
        tap_dy = tvec >> 1
        lp1 = lvec
        base_r = jnp.where(
            lvec == 0, _STARTS[0] * _BCELLS,
            jnp.where(lvec == 1, _STARTS[1] * _BCELLS,
                      jnp.where(lvec == 2, _STARTS[2] * _BCELLS,
                                _STARTS[3] * _BCELLS)))

        mask_x1 = tap_dx == 1
        mask_y1 = tap_dy == 1

        def load_and_fire(ci, idx_r, w_r, rows_r, sem):
            base = wid * ppw + ci * CHUNK
            pltpu.sync_copy(ts_hbm.at[pl.ds(base, CHUNK)], ts_v)
            pltpu.sync_copy(rho_hbm.at[pl.ds(base, CHUNK)], rho_v)

            def grp_body(g, carry2):
                off = g * L
                tsv = ts_v[pl.ds(off, L)]
                rhv = rho_v[pl.ds(off, L)]
                gx = 2.0 * jnp.minimum(jnp.maximum(rhv, 0.0), 1.0) - 1.0
                gy = 2.0 * jnp.minimum(jnp.maximum(tsv, 0.0), 1.0) - 1.0
                xsv = (gx + 1.0) * 0.5
                ysv = (gy + 1.0) * 0.5
                for kk in range(L):
                    x = jnp.broadcast_to(xsv[kk], (L,)) * wm1_f
                    y = jnp.broadcast_to(ysv[kk], (L,)) * hm1_f
                    x0 = jnp.minimum(x.astype(jnp.int32), wm2_i)
                    y0 = jnp.minimum(y.astype(jnp.int32), hm2_i)
                    wx = x - x0.astype(jnp.float32)
                    wy = y - y0.astype(jnp.float32)
                    xt = x0 + tap_dx
                    yt = y0 + tap_dy
                    blk = ((yt >> _LBH) << lp1) + (xt >> _LBW)
                    m = ((yt & (BH - 1)) << _LBW) + (xt & (BW - 1))
                    idx = (base_r + (blk << _LBC)
                           + ((m & (_Q - 1)) << 2) + (m >> _LQ))
                    wxx = jnp.where(mask_x1, wx, 1.0 - wx)
                    wyy = jnp.where(mask_y1, wy, 1.0 - wy)
                    poff = (off + kk) * NTAP
                    idx_r[pl.ds(poff, NTAP)] = idx
                    w_r[pl.ds(poff, NTAP)] = wxx * wyy
                return carry2

            lax.fori_loop(0, CHUNK // L, grp_body, 0)
            for j in range(NDMA):
                pltpu.async_copy(
                    tab_hbm.at[idx_r.at[pl.ds(j * 128, 128)]],
                    rows_r.at[pl.ds(j * 128, 128)], sem)

        def wait_gathers(idx_r, rows_r, sem):
            for j in range(NDMA):
                pltpu.make_async_copy(
                    tab_hbm.at[idx_r.at[pl.ds(j * 128, 128)]],
                    rows_r.at[pl.ds(j * 128, 128)], sem).wait()

        def accumulate(ci, w_r, rows_r, out_r):
            def pt_body(p, carry2):
                wvec = w_r[pl.ds(p * NTAP, NTAP)]
                for l in range(LEVELS):
                    acc_lo = None
                    acc_hi = None
                    for t in range(4):
                        j = 4 * l + t
                        v = rows_r[p * NTAP + j, :]
                        va = lax.bitcast_convert_type(v << 16, jnp.float32)
                        vb = lax.bitcast_convert_type(v & (-65536),
                                                      jnp.float32)
                        w = jnp.broadcast_to(wvec[j], (L,))
                        tlo = va * w
                        thi = vb * w
                        acc_lo = tlo if acc_lo is None else acc_lo + tlo
                        acc_hi = thi if acc_hi is None else acc_hi + thi
                    out_r[p, pl.ds(l * DIM, L)] = acc_lo
                    out_r[p, pl.ds(l * DIM + L, L)] = acc_hi
                return carry2

            lax.fori_loop(0, CHUNK, pt_body, 0)
            base = wid * ppw + ci * CHUNK
            pltpu.sync_copy(out_r, out_hbm.at[pl.ds(base, CHUNK)])

        npairs = nchunks // 2
        load_and_fire(0, idx_a, w_a, rows_a, sem_a)

        def pair_body(i2, carry):
            ci0 = i2 * 2
            load_and_fire(ci0 + 1, idx_b, w_b, rows_b, sem_b)
            wait_gathers(idx_a, rows_a, sem_a)
            accumulate(ci0, w_a, rows_a, out_a)

            @pl.when(i2 < npairs - 1)
            def _():
                load_and_fire(ci0 + 2, idx_a, w_a, rows_a, sem_a)

            wait_gathers(idx_b, rows_b, sem_b)
            accumulate(ci0 + 1, w_b, rows_b, out_b)
            return carry

        lax.fori_loop(0, npairs, pair_body, 0)

    return k(tsf, rhof, table)


def kernel(ts, rho, grid0, grid1, grid2, grid3):
    B, N = ts.shape
    P = B * N
    table = _build_table(grid0, grid1, grid2, grid3)
    out = _sc_sample(ts.reshape(P), rho.reshape(P), table)
    return out.reshape(B, N, LEVELS * DIM)


# speedup vs baseline: 2.5105x; 2.5105x over previous
import functools

import jax
import jax.numpy as jnp
from jax import lax
from jax.experimental import pallas as pl
from jax.experimental.pallas import tpu as pltpu
from jax.experimental.pallas import tpu_sc as plsc

DIM = 32
LEVELS = 4
H0, W0 = 64, 256
NC, NS, L = 2, 16, 16
NW = NC * NS
CHUNK = 64
NTAP = 4 * LEVELS
NDMA = NTAP * CHUNK // 128

BH, BW = 64, 256
_BCELLS = BH * BW
_BLOCKS = [(H0 << l) // BH * ((W0 << l) // BW) for l in range(LEVELS)]
_STARTS = [sum(_BLOCKS[:l]) for l in range(LEVELS)]
_TOTAL_BLOCKS = sum(_BLOCKS)
_ROWS = _TOTAL_BLOCKS * _BCELLS
_QS = _BCELLS // 8
_LBH = BH.bit_length() - 1
_LBW = BW.bit_length() - 1
_LBC = _BCELLS.bit_length() - 1
_LQS = _QS.bit_length() - 1


def _rne16(b):
    return (b + 0x7FFF + ((b >> 16) & 1)) >> 16


def _tr_body(g0, g1, g2, g3, out_ref):
    p = pl.program_id(0)
    a = jnp.where(
        p < _STARTS[1], g0[...],
        jnp.where(p < _STARTS[2], g1[...],
                  jnp.where(p < _STARTS[3], g2[...], g3[...])))
    af = a.reshape(DIM, _BCELLS)
    bl = lax.bitcast_convert_type(af[0:16], jnp.int32)
    bh = lax.bitcast_convert_type(af[16:32], jnp.int32)
    w = (_rne16(bh) << 16) | (_rne16(bl) & 0xFFFF)
    out_ref[...] = jnp.concatenate(
        [w[:, j * _QS:(j + 1) * _QS].T for j in range(8)], axis=1)


def _in_spec(l):
    nwb = (W0 << l) // BW

    def imap(p):
        q = jnp.clip(p - _STARTS[l], 0, _BLOCKS[l] - 1)
        return (0, q // nwb, q % nwb)

    return pl.BlockSpec((DIM, BH, BW), imap)


def _build_table(grid0, grid1, grid2, grid3):
    out = pl.pallas_call(
        _tr_body,
        grid=(_TOTAL_BLOCKS,),
        in_specs=[_in_spec(l) for l in range(LEVELS)],
        out_specs=pl.BlockSpec((_QS, 128), lambda p: (p, 0)),
        out_shape=jax.ShapeDtypeStruct((_TOTAL_BLOCKS * _QS, 128),
                                       jnp.int32),
    )(grid0[0], grid1[0], grid2[0], grid3[0])
    return out.reshape(_ROWS, DIM // 2)


def _sc_sample(tsf, rhof, table):
    P = tsf.shape[0]
    ppw = P // NW
    nchunks = ppw // CHUNK
    mesh = plsc.VectorSubcoreMesh(core_axis_name="c", subcore_axis_name="s")

    @functools.partial(
        pl.kernel,
        out_type=jax.ShapeDtypeStruct((P, LEVELS * DIM), jnp.float32),
        mesh=mesh,
        scratch_types=[
            pltpu.VMEM((CHUNK,), jnp.float32),
            pltpu.VMEM((CHUNK,), jnp.float32),
            pltpu.VMEM((NTAP * CHUNK,), jnp.int32),
            pltpu.VMEM((NTAP * CHUNK,), jnp.int32),
            pltpu.VMEM((NTAP * CHUNK,), jnp.float32),
            pltpu.VMEM((NTAP * CHUNK,), jnp.float32),
            pltpu.VMEM((NTAP * CHUNK, DIM // 2), jnp.int32),
            pltpu.VMEM((NTAP * CHUNK, DIM // 2), jnp.int32),
            pltpu.VMEM((CHUNK, LEVELS * DIM), jnp.float32),
            pltpu.VMEM((CHUNK, LEVELS * DIM), jnp.float32),
            pltpu.SemaphoreType.DMA,
            pltpu.SemaphoreType.DMA,
        ],
        compiler_params=pltpu.CompilerParams(use_tc_tiling_on_sc=False),
    )
    def k(ts_hbm, rho_hbm, tab_hbm, out_hbm,
          ts_v, rho_v, idx_a, idx_b, w_a, w_b, rows_a, rows_b,
          out_a, out_b, sem_a, sem_b):
        wid = lax.axis_index("s") * NC + lax.axis_index("c")

        lane = lax.iota(jnp.int32, L)
        tvec = lane & 3
        lvec = lane >> 2
        wl_i = W0 << lvec
        hl_i = H0 << lvec
        wm1_f = (wl_i - 1).astype(jnp.float32)
        hm1_f = (hl_i - 1).astype(jnp.float32)
        wm2_i = wl_i - 2
        hm2_i = hl_i - 2
        tap_dx = tvec & 1
        tap_dy = tvec >> 1
        lp1 = lvec
        base_r = jnp.where(
            lvec == 0, _STARTS[0] * _BCELLS,
            jnp.where(lvec == 1, _STARTS[1] * _BCELLS,
                      jnp.where(lvec == 2, _STARTS[2] * _BCELLS,
                                _STARTS[3] * _BCELLS)))

        mask_x1 = tap_dx == 1
        mask_y1 = tap_dy == 1

        def load_and_fire(ci, idx_r, w_r, rows_r, sem):
            base = wid * ppw + ci * CHUNK
            pltpu.sync_copy(ts_hbm.at[pl.ds(base, CHUNK)], ts_v)
            pltpu.sync_copy(rho_hbm.at[pl.ds(base, CHUNK)], rho_v)

            def grp_body(g, carry2):
                off = g * L
                tsv = ts_v[pl.ds(off, L)]
                rhv = rho_v[pl.ds(off, L)]
                gx = 2.0 * jnp.minimum(jnp.maximum(rhv, 0.0), 1.0) - 1.0
                gy = 2.0 * jnp.minimum(jnp.maximum(tsv, 0.0), 1.0) - 1.0
                xsv = (gx + 1.0) * 0.5
                ysv = (gy + 1.0) * 0.5
                for kk in range(L):
                    x = jnp.broadcast_to(xsv[kk], (L,)) * wm1_f
                    y = jnp.broadcast_to(ysv[kk], (L,)) * hm1_f
                    x0 = jnp.minimum(x.astype(jnp.int32), wm2_i)
                    y0 = jnp.minimum(y.astype(jnp.int32), hm2_i)
                    wx = x - x0.astype(jnp.float32)
                    wy = y - y0.astype(jnp.float32)
                    xt = x0 + tap_dx
                    yt = y0 + tap_dy
                    blk = ((yt >> _LBH) << lp1) + (xt >> _LBW)
                    m = ((yt & (BH - 1)) << _LBW) + (xt & (BW - 1))
                    idx = (base_r + (blk << _LBC)
                           + ((m & (_QS - 1)) << 3) + (m >> _LQS))
                    wxx = jnp.where(mask_x1, wx, 1.0 - wx)
                    wyy = jnp.where(mask_y1, wy, 1.0 - wy)
                    poff = (off + kk) * NTAP
                    idx_r[pl.ds(poff, NTAP)] = idx
                    w_r[pl.ds(poff, NTAP)] = wxx * wyy
                return carry2

            lax.fori_loop(0, CHUNK // L, grp_body, 0)
            for j in range(NDMA):
                pltpu.async_copy(
                    tab_hbm.at[idx_r.at[pl.ds(j * 128, 128)]],
                    rows_r.at[pl.ds(j * 128, 128)], sem)

        def wait_gathers(idx_r, rows_r, sem):
            for j in range(NDMA):
                pltpu.make_async_copy(
                    tab_hbm.at[idx_r.at[pl.ds(j * 128, 128)]],
                    rows_r.at[pl.ds(j * 128, 128)], sem).wait()

        def accumulate(ci, w_r, rows_r, out_r):
            def pt_body(p, carry2):
                wvec = w_r[pl.ds(p * NTAP, NTAP)]
                for l in range(LEVELS):
                    acc_lo = None
                    acc_hi = None
                    for t in range(4):
                        j = 4 * l + t
                        v = rows_r[p * NTAP + j, :]
                        va = lax.bitcast_convert_type(v << 16, jnp.float32)
                        vb = lax.bitcast_convert_type(v & (-65536),
                                                      jnp.float32)
                        w = jnp.broadcast_to(wvec[j], (L,))
                        tlo = va * w
                        thi = vb * w
                        acc_lo = tlo if acc_lo is None else acc_lo + tlo
                        acc_hi = thi if acc_hi is None else acc_hi + thi
                    out_r[p, pl.ds(l * DIM, L)] = acc_lo
                    out_r[p, pl.ds(l * DIM + L, L)] = acc_hi
                return carry2

            lax.fori_loop(0, CHUNK, pt_body, 0)
            base = wid * ppw + ci * CHUNK
            pltpu.sync_copy(out_r, out_hbm.at[pl.ds(base, CHUNK)])

        npairs = nchunks // 2
        load_and_fire(0, idx_a, w_a, rows_a, sem_a)

        def pair_body(i2, carry):
            ci0 = i2 * 2
            load_and_fire(ci0 + 1, idx_b, w_b, rows_b, sem_b)
            wait_gathers(idx_a, rows_a, sem_a)
            accumulate(ci0, w_a, rows_a, out_a)

            @pl.when(i2 < npairs - 1)
            def _():
                load_and_fire(ci0 + 2, idx_a, w_a, rows_a, sem_a)

            wait_gathers(idx_b, rows_b, sem_b)
            accumulate(ci0 + 1, w_b, rows_b, out_b)
            return carry

        lax.fori_loop(0, npairs, pair_body, 0)

    return k(tsf, rhof, table)


def kernel(ts, rho, grid0, grid1, grid2, grid3):
    B, N = ts.shape
    P = B * N
    table = _build_table(grid0, grid1, grid2, grid3)
    out = _sc_sample(ts.reshape(P), rho.reshape(P), table)
    return out.reshape(B, N, LEVELS * DIM)


# speedup vs baseline: 3.5287x; 1.4055x over previous
import functools

import jax
import jax.numpy as jnp
from jax import lax
from jax.experimental import pallas as pl
from jax.experimental.pallas import tpu as pltpu
from jax.experimental.pallas import tpu_sc as plsc

DIM = 32
LEVELS = 4
H0, W0 = 64, 256
NC, NS, L = 2, 16, 16
NW = NC * NS
CHUNK = 64
NTAP = 4 * LEVELS
NDMA = NTAP * CHUNK // 128

BH, BW = 64, 256
_BCELLS = BH * BW
_BLOCKS = [(H0 << l) // BH * ((W0 << l) // BW) for l in range(LEVELS)]
_STARTS = [sum(_BLOCKS[:l]) for l in range(LEVELS)]
_TOTAL_BLOCKS = sum(_BLOCKS)
_ROWS = _TOTAL_BLOCKS * _BCELLS
_QS = _BCELLS // 8
_LBH = BH.bit_length() - 1
_LBW = BW.bit_length() - 1
_LBC = _BCELLS.bit_length() - 1
_LQS = _QS.bit_length() - 1


def _rne16(b):
    return (b + 0x7FFF + ((b >> 16) & 1)) >> 16


def _tr_body(g0, g1, g2, g3, out_ref):
    p = pl.program_id(0)
    a = jnp.where(
        p < _STARTS[1], g0[...],
        jnp.where(p < _STARTS[2], g1[...],
                  jnp.where(p < _STARTS[3], g2[...], g3[...])))
    af = a.reshape(DIM, _BCELLS)
    bl = lax.bitcast_convert_type(af[0:16], jnp.int32)
    bh = lax.bitcast_convert_type(af[16:32], jnp.int32)
    w = (_rne16(bh) << 16) | (_rne16(bl) & 0xFFFF)
    wcat = jnp.concatenate(
        [w[:, 0:_BCELLS // 2], w[:, _BCELLS // 2:_BCELLS]], axis=0)
    out_ref[...] = jnp.concatenate(
        [wcat[:, j * _QS:(j + 1) * _QS].T for j in range(4)], axis=1)


def _in_spec(l):
    nwb = (W0 << l) // BW

    def imap(p):
        q = jnp.clip(p - _STARTS[l], 0, _BLOCKS[l] - 1)
        return (0, q // nwb, q % nwb)

    return pl.BlockSpec((DIM, BH, BW), imap)


def _build_table(grid0, grid1, grid2, grid3):
    out = pl.pallas_call(
        _tr_body,
        grid=(_TOTAL_BLOCKS,),
        in_specs=[_in_spec(l) for l in range(LEVELS)],
        out_specs=pl.BlockSpec((_QS, 128), lambda p: (p, 0)),
        out_shape=jax.ShapeDtypeStruct((_TOTAL_BLOCKS * _QS, 128),
                                       jnp.int32),
    )(grid0[0], grid1[0], grid2[0], grid3[0])
    return out.reshape(_ROWS, DIM // 2)


def _sc_sample(tsf, rhof, table):
    P = tsf.shape[0]
    ppw = P // NW
    nchunks = ppw // CHUNK
    mesh = plsc.VectorSubcoreMesh(core_axis_name="c", subcore_axis_name="s")

    @functools.partial(
        pl.kernel,
        out_type=jax.ShapeDtypeStruct((P, LEVELS * DIM), jnp.float32),
        mesh=mesh,
        scratch_types=[
            pltpu.VMEM((CHUNK,), jnp.float32),
            pltpu.VMEM((CHUNK,), jnp.float32),
            pltpu.VMEM((NTAP * CHUNK,), jnp.int32),
            pltpu.VMEM((NTAP * CHUNK,), jnp.int32),
            pltpu.VMEM((NTAP * CHUNK,), jnp.float32),
            pltpu.VMEM((NTAP * CHUNK,), jnp.float32),
            pltpu.VMEM((NTAP * CHUNK, DIM // 2), jnp.int32),
            pltpu.VMEM((NTAP * CHUNK, DIM // 2), jnp.int32),
            pltpu.VMEM((CHUNK, LEVELS * DIM), jnp.float32),
            pltpu.VMEM((CHUNK, LEVELS * DIM), jnp.float32),
            pltpu.SemaphoreType.DMA,
            pltpu.SemaphoreType.DMA,
        ],
        compiler_params=pltpu.CompilerParams(use_tc_tiling_on_sc=False),
    )
    def k(ts_hbm, rho_hbm, tab_hbm, out_hbm,
          ts_v, rho_v, idx_a, idx_b, w_a, w_b, rows_a, rows_b,
          out_a, out_b, sem_a, sem_b):
        wid = lax.axis_index("s") * NC + lax.axis_index("c")

        lane = lax.iota(jnp.int32, L)
        tvec = lane & 3
        lvec = lane >> 2
        wl_i = W0 << lvec
        hl_i = H0 << lvec
        wm1_f = (wl_i - 1).astype(jnp.float32)
        hm1_f = (hl_i - 1).astype(jnp.float32)
        wm2_i = wl_i - 2
        hm2_i = hl_i - 2
        tap_dx = tvec & 1
        tap_dy = tvec >> 1
        lp1 = lvec
        base_r = jnp.where(
            lvec == 0, _STARTS[0] * _BCELLS,
            jnp.where(lvec == 1, _STARTS[1] * _BCELLS,
                      jnp.where(lvec == 2, _STARTS[2] * _BCELLS,
                                _STARTS[3] * _BCELLS)))

        mask_x1 = tap_dx == 1
        mask_y1 = tap_dy == 1

        def load_and_fire(ci, idx_r, w_r, rows_r, sem):
            base = wid * ppw + ci * CHUNK
            pltpu.sync_copy(ts_hbm.at[pl.ds(base, CHUNK)], ts_v)
            pltpu.sync_copy(rho_hbm.at[pl.ds(base, CHUNK)], rho_v)

            def grp_body(g, carry2):
                off = g * L
                tsv = ts_v[pl.ds(off, L)]
                rhv = rho_v[pl.ds(off, L)]
                gx = 2.0 * jnp.minimum(jnp.maximum(rhv, 0.0), 1.0) - 1.0
                gy = 2.0 * jnp.minimum(jnp.maximum(tsv, 0.0), 1.0) - 1.0
                xsv = (gx + 1.0) * 0.5
                ysv = (gy + 1.0) * 0.5
                for kk in range(L):
                    x = jnp.broadcast_to(xsv[kk], (L,)) * wm1_f
                    y = jnp.broadcast_to(ysv[kk], (L,)) * hm1_f
                    x0 = jnp.minimum(x.astype(jnp.int32), wm2_i)
                    y0 = jnp.minimum(y.astype(jnp.int32), hm2_i)
                    wx = x - x0.astype(jnp.float32)
                    wy = y - y0.astype(jnp.float32)
                    xt = x0 + tap_dx
                    yt = y0 + tap_dy
                    blk = ((yt >> _LBH) << lp1) + (xt >> _LBW)
                    m = ((yt & (BH - 1)) << _LBW) + (xt & (BW - 1))
                    mm = m & (_BCELLS // 2 - 1)
                    idx = (base_r + (blk << _LBC)
                           + ((mm & (_QS - 1)) << 3)
                           + ((mm >> _LQS) << 1) + (m >> (_LBC - 1)))
                    wxx = jnp.where(mask_x1, wx, 1.0 - wx)
                    wyy = jnp.where(mask_y1, wy, 1.0 - wy)
                    poff = (off + kk) * NTAP
                    idx_r[pl.ds(poff, NTAP)] = idx
                    w_r[pl.ds(poff, NTAP)] = wxx * wyy
                return carry2

            lax.fori_loop(0, CHUNK // L, grp_body, 0)
            for j in range(NDMA):
                pltpu.async_copy(
                    tab_hbm.at[idx_r.at[pl.ds(j * 128, 128)]],
                    rows_r.at[pl.ds(j * 128, 128)], sem)

        def wait_gathers(idx_r, rows_r, sem):
            for j in range(NDMA):
                pltpu.make_async_copy(
                    tab_hbm.at[idx_r.at[pl.ds(j * 128, 128)]],
                    rows_r.at[pl.ds(j * 128, 128)], sem).wait()

        def accumulate(ci, w_r, rows_r, out_r):
            def pt_body(p, carry2):
                wvec = w_r[pl.ds(p * NTAP, NTAP)]
                for l in range(LEVELS):
                    acc_lo = None
                    acc_hi = None
                    for t in range(4):
                        j = 4 * l + t
                        v = rows_r[p * NTAP + j, :]
                        va = lax.bitcast_convert_type(v << 16, jnp.float32)
                        vb = lax.bitcast_convert_type(v & (-65536),
                                                      jnp.float32)
                        w = jnp.broadcast_to(wvec[j], (L,))
                        tlo = va * w
                        thi = vb * w
                        acc_lo = tlo if acc_lo is None else acc_lo + tlo
                        acc_hi = thi if acc_hi is None else acc_hi + thi
                    out_r[p, pl.ds(l * DIM, L)] = acc_lo
                    out_r[p, pl.ds(l * DIM + L, L)] = acc_hi
                return carry2

            lax.fori_loop(0, CHUNK, pt_body, 0)
            base = wid * ppw + ci * CHUNK
            pltpu.sync_copy(out_r, out_hbm.at[pl.ds(base, CHUNK)])

        npairs = nchunks // 2
        load_and_fire(0, idx_a, w_a, rows_a, sem_a)

        def pair_body(i2, carry):
            ci0 = i2 * 2
            load_and_fire(ci0 + 1, idx_b, w_b, rows_b, sem_b)
            wait_gathers(idx_a, rows_a, sem_a)
            accumulate(ci0, w_a, rows_a, out_a)

            @pl.when(i2 < npairs - 1)
            def _():
                load_and_fire(ci0 + 2, idx_a, w_a, rows_a, sem_a)

            wait_gathers(idx_b, rows_b, sem_b)
            accumulate(ci0 + 1, w_b, rows_b, out_b)
            return carry

        lax.fori_loop(0, npairs, pair_body, 0)

    return k(tsf, rhof, table)


def kernel(ts, rho, grid0, grid1, grid2, grid3):
    B, N = ts.shape
    P = B * N
    table = _build_table(grid0, grid1, grid2, grid3)
    out = _sc_sample(ts.reshape(P), rho.reshape(P), table)
    return out.reshape(B, N, LEVELS * DIM)
